# baseline (speedup 1.0000x reference)
import functools

import jax
import jax.numpy as jnp
from jax import lax
from jax.experimental import pallas as pl
from jax.experimental.pallas import tpu as pltpu
from jax.experimental.pallas import tpu_sc as plsc

Q, D, K = 4096, 16, 1024
L = 16
NC, NS = 2, 16
NW = NC * NS
QPW = Q // NW
CHUNKS = K // L
QBLK = 1
CUNROLL = 8

_mesh = plsc.VectorSubcoreMesh(core_axis_name="c", subcore_axis_name="s")


@functools.partial(
    pl.kernel,
    out_type=[
        jax.ShapeDtypeStruct((Q,), jnp.int32),
        jax.ShapeDtypeStruct((NW, L), jnp.float32),
    ],
    mesh=_mesh,
    compiler_params=pltpu.CompilerParams(needs_layout_passes=False),
    scratch_types=[
        pltpu.VMEM((QPW, D), jnp.float32),
        pltpu.VMEM((K * D,), jnp.float32),
        pltpu.VMEM((D, K), jnp.float32),
        pltpu.VMEM((QPW,), jnp.int32),
        pltpu.VMEM((L,), jnp.float32),
    ],
)
def _sc_kernel(e_hbm, c_hbm, assign_hbm, part_hbm,
               e_v, c_v, ct_v, idx_v, pv_v):
    cid = lax.axis_index("c")
    sid = lax.axis_index("s")
    wid = sid * NC + cid
    base = wid * QPW

    pltpu.sync_copy(e_hbm.at[pl.ds(base, QPW), :], e_v)
    pltpu.sync_copy(c_hbm, c_v)

    iota = lax.iota(jnp.int32, L)
    gdn = lax.GatherDimensionNumbers(
        offset_dims=(), collapsed_slice_dims=(0,), start_index_map=(0,))

    def lane_splat(vec, d):
        idx = jnp.full((L, 1), d, jnp.int32)
        return lax.gather(vec, idx, gdn, slice_sizes=(1,),
                          mode=lax.GatherScatterMode.PROMISE_IN_BOUNDS)

    iota_d = iota * D
    for c in range(CHUNKS):
        rows_d = iota_d + c * L * D
        for d in range(D):
            col = plsc.load_gather(c_v, [rows_d + d])
            ct_v[d, pl.ds(c * L, L)] = col

    neg = jnp.full((L,), -1.0, jnp.float32)
    zero_i = jnp.zeros((L,), jnp.int32)
    big_i = jnp.full((L,), K, jnp.int32)
    zero_f = jnp.zeros((L,), jnp.float32)
    GPS = L // QBLK

    def q_group(g, outer_carry):
        lacc, idxvec = outer_carry
        qs = [g * QBLK + j for j in range(QBLK)]
        splats = []
        for q in qs:
            ev = e_v[q, :]
            splats.append([lane_splat(ev, d) for d in range(D)])

        def chunk_oct(c8, carry):
            mvs = list(carry[0])
            mis = list(carry[1])
            for cc in range(CUNROLL):
                cbase = (c8 * CUNROLL + cc) * L
                idxc = iota + cbase
                cts = [ct_v[d, pl.ds(cbase, L)] for d in range(D)]
                for j in range(QBLK):
                    a = None
                    for d in range(D):
                        diff = cts[d] - splats[j][d]
                        sq = diff * diff
                        a = sq if a is None else a + sq
                    m = a > mvs[j]
                    mvs[j] = jnp.where(m, a, mvs[j])
                    mis[j] = jnp.where(m, idxc, mis[j])
            return (tuple(mvs), tuple(mis))

        carry0 = (tuple(neg for _ in range(QBLK)),
                  tuple(zero_i for _ in range(QBLK)))
        mvs, mis = lax.fori_loop(0, CHUNKS // CUNROLL, chunk_oct, carry0)

        for j in range(QBLK):
            jj = (g % GPS) * QBLK + j
            maxd = jnp.max(mvs[j])
            lacc = lacc + jnp.where(iota == jj, maxd, zero_f)
            cand = jnp.where(mvs[j] == maxd, mis[j], big_i)
            idxvec = jnp.where(iota == jj, jnp.min(cand), idxvec)

        @pl.when(g % GPS == GPS - 1)
        def _store():
            idx_v[pl.ds((g // GPS) * L, L)] = idxvec

        return (lacc, idxvec)

    lacc, _ = lax.fori_loop(0, QPW // QBLK, q_group, (zero_f, zero_i))

    total = jnp.sum(lacc)
    pv_v[...] = jnp.where(iota == 0, total, zero_f)

    pltpu.sync_copy(idx_v, assign_hbm.at[pl.ds(base, QPW)])
    pltpu.sync_copy(pv_v, part_hbm.at[wid])


def kernel(embeddings, centroids):
    assignments, partials = _sc_kernel(embeddings, centroids.reshape(-1))
    loss = jnp.sum(partials)
    return (loss, assignments)


# speedup vs baseline: 4.3031x; 4.3031x over previous
import functools

import jax
import jax.numpy as jnp
from jax import lax
from jax.experimental import pallas as pl
from jax.experimental.pallas import tpu as pltpu
from jax.experimental.pallas import tpu_sc as plsc

Q, D, K = 4096, 16, 1024
L = 16
NC, NS = 2, 16
NW = NC * NS
QPW = Q // NW
CHUNKS = K // L
QB = 16
NBLK = QPW // QB

TQ = 256
TGRID = Q // TQ


def _tc_dist_body(e_ref, ct_ref, s_ref):
    acc = None
    for d in range(D):
        ecol = e_ref[:, d:d + 1]
        crow = ct_ref[d:d + 1, :]
        diff = ecol - crow
        sq = diff * diff
        acc = sq if acc is None else acc + sq
    s_ref[...] = acc


_tc_dist = pl.pallas_call(
    _tc_dist_body,
    grid=(TGRID,),
    in_specs=[
        pl.BlockSpec((TQ, D), lambda i: (i, 0)),
        pl.BlockSpec((D, K), lambda i: (0, 0)),
    ],
    out_specs=pl.BlockSpec((TQ, K), lambda i: (i, 0)),
    out_shape=jax.ShapeDtypeStruct((Q, K), jnp.float32),
    compiler_params=pltpu.CompilerParams(
        dimension_semantics=("arbitrary",)),
)


_mesh = plsc.VectorSubcoreMesh(core_axis_name="c", subcore_axis_name="s")


@functools.partial(
    pl.kernel,
    out_type=[
        jax.ShapeDtypeStruct((Q,), jnp.int32),
        jax.ShapeDtypeStruct((NW, L), jnp.float32),
    ],
    mesh=_mesh,
    compiler_params=pltpu.CompilerParams(needs_layout_passes=False),
    scratch_types=[
        pltpu.VMEM((2, QB, K), jnp.float32),
        pltpu.VMEM((QPW,), jnp.int32),
        pltpu.VMEM((L,), jnp.float32),
        pltpu.SemaphoreType.DMA,
        pltpu.SemaphoreType.DMA,
    ],
)
def _sc_argmax(s_hbm, assign_hbm, part_hbm, s_buf, idx_v, pv_v, sem0, sem1):
    cid = lax.axis_index("c")
    sid = lax.axis_index("s")
    wid = sid * NC + cid
    base = wid * QPW

    sems = (sem0, sem1)
    iota = lax.iota(jnp.int32, L)
    neg = jnp.full((L,), -1.0, jnp.float32)
    zero_i = jnp.zeros((L,), jnp.int32)
    zero_f = jnp.zeros((L,), jnp.float32)
    big_i = jnp.full((L,), K, jnp.int32)

    def start(b):
        slot = b % 2
        return pltpu.async_copy(
            s_hbm.at[pl.ds(base + b * QB, QB), :], s_buf.at[slot], sems[slot])

    h = {0: start(0)}
    lacc = zero_f
    for b in range(NBLK):
        slot = b % 2
        if b + 1 < NBLK:
            h[b + 1] = start(b + 1)
        h[b].wait()

        def row(jj, carry):
            la, idxvec = carry
            mv = neg
            mi = zero_i
            for c in range(CHUNKS):
                sv = s_buf[slot, jj, pl.ds(c * L, L)]
                m = sv > mv
                mv = jnp.where(m, sv, mv)
                mi = jnp.where(m, jnp.full((L,), c, jnp.int32), mi)
            maxd = jnp.max(mv)
            cand = jnp.where(mv == maxd, mi * L + iota, big_i)
            la = la + jnp.where(iota == jj, maxd, zero_f)
            idxvec = jnp.where(iota == jj, jnp.min(cand), idxvec)
            return (la, idxvec)

        lacc, idxvec = lax.fori_loop(0, QB, row, (lacc, zero_i))
        idx_v[pl.ds(b * QB, QB)] = idxvec

    total = jnp.sum(lacc)
    pv_v[...] = jnp.where(iota == 0, total, zero_f)

    pltpu.sync_copy(idx_v, assign_hbm.at[pl.ds(base, QPW)])
    pltpu.sync_copy(pv_v, part_hbm.at[wid])


def kernel(embeddings, centroids):
    s = _tc_dist(embeddings, centroids.T)
    assignments, partials = _sc_argmax(s)
    loss = jnp.sum(partials)
    return (loss, assignments)


# speedup vs baseline: 5.6813x; 1.3203x over previous
import functools

import jax
import jax.numpy as jnp
from jax import lax
from jax.experimental import pallas as pl
from jax.experimental.pallas import tpu as pltpu
from jax.experimental.pallas import tpu_sc as plsc

Q, D, K = 4096, 16, 1024
L = 16
NC, NS = 2, 16
NW = NC * NS
QPW = Q // NW
CHUNKS = K // L
QB = 16
NBLK = QPW // QB

TQ = 256
TGRID = Q // TQ


def _tc_dist_body(e_ref, ct_ref, s_ref):
    acc = None
    for d in range(D):
        ecol = e_ref[:, d:d + 1]
        crow = ct_ref[d:d + 1, :]
        diff = ecol - crow
        sq = diff * diff
        acc = sq if acc is None else acc + sq
    s_ref[...] = acc


_tc_dist = pl.pallas_call(
    _tc_dist_body,
    grid=(TGRID,),
    in_specs=[
        pl.BlockSpec((TQ, D), lambda i: (i, 0)),
        pl.BlockSpec((D, K), lambda i: (0, 0)),
    ],
    out_specs=pl.BlockSpec((TQ, K), lambda i: (i, 0)),
    out_shape=jax.ShapeDtypeStruct((Q, K), jnp.float32),
    compiler_params=pltpu.CompilerParams(
        dimension_semantics=("arbitrary",),
        allow_input_fusion=[False, True]),
)


_mesh = plsc.VectorSubcoreMesh(core_axis_name="c", subcore_axis_name="s")


@functools.partial(
    pl.kernel,
    out_type=[
        jax.ShapeDtypeStruct((Q,), jnp.int32),
        jax.ShapeDtypeStruct((NW, L), jnp.float32),
    ],
    mesh=_mesh,
    compiler_params=pltpu.CompilerParams(needs_layout_passes=False),
    scratch_types=[
        pltpu.VMEM((2, QB, K), jnp.float32),
        pltpu.VMEM((QPW,), jnp.int32),
        pltpu.VMEM((L,), jnp.float32),
        pltpu.SemaphoreType.DMA,
        pltpu.SemaphoreType.DMA,
    ],
)
def _sc_argmax(s_hbm, assign_hbm, part_hbm, s_buf, idx_v, pv_v, sem0, sem1):
    cid = lax.axis_index("c")
    sid = lax.axis_index("s")
    wid = sid * NC + cid
    base = wid * QPW

    sems = (sem0, sem1)
    iota = lax.iota(jnp.int32, L)
    neg = jnp.full((L,), -1.0, jnp.float32)
    zero_i = jnp.zeros((L,), jnp.int32)
    zero_f = jnp.zeros((L,), jnp.float32)
    big_i = jnp.full((L,), K, jnp.int32)

    def start(b):
        slot = b % 2
        return pltpu.async_copy(
            s_hbm.at[pl.ds(base + b * QB, QB), :], s_buf.at[slot], sems[slot])

    h = {0: start(0)}
    lacc = zero_f
    for b in range(NBLK):
        slot = b % 2
        if b + 1 < NBLK:
            h[b + 1] = start(b + 1)
        h[b].wait()

        def row(jj, carry):
            la, idxvec = carry
            NG = 4
            GC = CHUNKS // NG
            mvs = [neg] * NG
            mis = [zero_i] * NG
            for g in range(NG):
                for cc in range(GC):
                    c = g * GC + cc
                    sv = s_buf[slot, jj, pl.ds(c * L, L)]
                    m = sv > mvs[g]
                    mvs[g] = jnp.where(m, sv, mvs[g])
                    mis[g] = jnp.where(m, jnp.full((L,), c, jnp.int32),
                                       mis[g])
            mv = mvs[0]
            mi = mis[0]
            for g in range(1, NG):
                m = mvs[g] > mv
                mv = jnp.where(m, mvs[g], mv)
                mi = jnp.where(m, mis[g], mi)
            maxd = jnp.max(mv)
            cand = jnp.where(mv == maxd, mi * L + iota, big_i)
            la = la + jnp.where(iota == jj, maxd, zero_f)
            idxvec = jnp.where(iota == jj, jnp.min(cand), idxvec)
            return (la, idxvec)

        lacc, idxvec = lax.fori_loop(0, QB, row, (lacc, zero_i))
        idx_v[pl.ds(b * QB, QB)] = idxvec

    total = jnp.sum(lacc)
    pv_v[...] = jnp.where(iota == 0, total, zero_f)

    pltpu.sync_copy(idx_v, assign_hbm.at[pl.ds(base, QPW)])
    pltpu.sync_copy(pv_v, part_hbm.at[wid])


def kernel(embeddings, centroids):
    s = _tc_dist(embeddings, centroids.T)
    assignments, partials = _sc_argmax(s)
    loss = jnp.sum(partials)
    return (loss, assignments)


# speedup vs baseline: 5.6992x; 1.0032x over previous
import functools

import jax
import jax.numpy as jnp
from jax import lax
from jax.experimental import pallas as pl
from jax.experimental.pallas import tpu as pltpu
from jax.experimental.pallas import tpu_sc as plsc

Q, D, K = 4096, 16, 1024
L = 16
NC, NS = 2, 16
NW = NC * NS
CHUNKS = K // L
QB = 16
RING = 4

NSPLIT = 2
QH = Q // NSPLIT

TQ = 256


def _tc_dist_body(e_ref, ct_ref, s_ref):
    acc = None
    for d in range(D):
        ecol = e_ref[:, d:d + 1]
        crow = ct_ref[d:d + 1, :]
        diff = ecol - crow
        sq = diff * diff
        acc = sq if acc is None else acc + sq
    s_ref[...] = acc


def _make_tc(nq):
    return pl.pallas_call(
        _tc_dist_body,
        grid=(nq // TQ,),
        in_specs=[
            pl.BlockSpec((TQ, D), lambda i: (i, 0)),
            pl.BlockSpec((D, K), lambda i: (0, 0)),
        ],
        out_specs=pl.BlockSpec((TQ, K), lambda i: (i, 0)),
        out_shape=jax.ShapeDtypeStruct((nq, K), jnp.float32),
        compiler_params=pltpu.CompilerParams(
            dimension_semantics=("arbitrary",),
            allow_input_fusion=[False, True]),
    )


_mesh = plsc.VectorSubcoreMesh(core_axis_name="c", subcore_axis_name="s")


def _make_sc(nq):
    qpw = nq // NW
    nblk = qpw // QB
    ring = min(RING, nblk)

    @functools.partial(
        pl.kernel,
        out_type=[
            jax.ShapeDtypeStruct((nq,), jnp.int32),
            jax.ShapeDtypeStruct((NW, L), jnp.float32),
        ],
        mesh=_mesh,
        compiler_params=pltpu.CompilerParams(needs_layout_passes=False),
        scratch_types=[
            pltpu.VMEM((ring, QB, K), jnp.float32),
            pltpu.VMEM((qpw,), jnp.int32),
            pltpu.VMEM((L,), jnp.float32),
        ] + [pltpu.SemaphoreType.DMA] * ring,
    )
    def sc_argmax(s_hbm, assign_hbm, part_hbm, s_buf, idx_v, pv_v, *sems):
        cid = lax.axis_index("c")
        sid = lax.axis_index("s")
        wid = sid * NC + cid
        base = wid * qpw

        iota = lax.iota(jnp.int32, L)
        neg = jnp.full((L,), -1.0, jnp.float32)
        zero_i = jnp.zeros((L,), jnp.int32)
        zero_f = jnp.zeros((L,), jnp.float32)
        big_i = jnp.full((L,), K, jnp.int32)

        def start(b):
            slot = b % ring
            return pltpu.async_copy(
                s_hbm.at[pl.ds(base + b * QB, QB), :], s_buf.at[slot],
                sems[slot])

        h = {}
        for b in range(ring):
            h[b] = start(b)
        lacc = zero_f
        for b in range(nblk):
            slot = b % ring
            h[b].wait()

            def row(jj, carry):
                la, idxvec = carry
                NG = 4
                GC = CHUNKS // NG
                mvs = [neg] * NG
                mis = [zero_i] * NG
                for g in range(NG):
                    for cc in range(GC):
                        c = g * GC + cc
                        sv = s_buf[slot, jj, pl.ds(c * L, L)]
                        m = sv > mvs[g]
                        mvs[g] = jnp.where(m, sv, mvs[g])
                        mis[g] = jnp.where(m, jnp.full((L,), c, jnp.int32),
                                           mis[g])
                mv = mvs[0]
                mi = mis[0]
                for g in range(1, NG):
                    m = mvs[g] > mv
                    mv = jnp.where(m, mvs[g], mv)
                    mi = jnp.where(m, mis[g], mi)
                maxd = jnp.max(mv)
                cand = jnp.where(mv == maxd, mi * L + iota, big_i)
                la = la + jnp.where(iota == jj, maxd, zero_f)
                idxvec = jnp.where(iota == jj, jnp.min(cand), idxvec)
                return (la, idxvec)

            lacc, idxvec = lax.fori_loop(0, QB, row, (lacc, zero_i))
            idx_v[pl.ds(b * QB, QB)] = idxvec
            if b + ring < nblk:
                h[b + ring] = start(b + ring)

        total = jnp.sum(lacc)
        pv_v[...] = jnp.where(iota == 0, total, zero_f)

        pltpu.sync_copy(idx_v, assign_hbm.at[pl.ds(base, qpw)])
        pltpu.sync_copy(pv_v, part_hbm.at[wid])

    return sc_argmax


_tc_half = _make_tc(QH)
_sc_half = _make_sc(QH)


def kernel(embeddings, centroids):
    ct = centroids.T
    halves = []
    for i in range(NSPLIT):
        s = _tc_half(embeddings[i * QH:(i + 1) * QH], ct)
        halves.append(_sc_half(s))
    assignments = jnp.concatenate([a for a, _ in halves])
    loss = jnp.sum(jnp.stack([p for _, p in halves]))
    return (loss, assignments)
